# baseline (speedup 1.0000x reference)
import functools

import jax
import jax.numpy as jnp
import numpy as np
from jax import lax
from jax.experimental import pallas as pl
from jax.experimental.pallas import tpu as pltpu
from jax.experimental.pallas import tpu_sc as plsc

LEVELS = 4
BATCH = 128
D = 1024
NPAIR = 8128
P_EXP = 2

NC = 2
NS = 16
NW = NC * NS
LANES = 16

C = 32
CHUNKS_PER_LEVEL = NPAIR // C
ROUNDS = -(-CHUNKS_PER_LEVEL // NW)

_i0, _i1 = np.triu_indices(n=BATCH, k=1)
G0 = np.concatenate([l * BATCH + _i0 for l in range(LEVELS)]).astype(np.int32)
G1 = np.concatenate([l * BATCH + _i1 for l in range(LEVELS)]).astype(np.int32)

_mesh = plsc.VectorSubcoreMesh(core_axis_name="c", subcore_axis_name="s")


@functools.partial(
    pl.kernel,
    mesh=_mesh,
    out_type=[jax.ShapeDtypeStruct((NPAIR, D), jnp.float32)
              for _ in range(LEVELS)],
    scratch_types=[
        pltpu.VMEM((C,), jnp.int32),
        pltpu.VMEM((C,), jnp.int32),
        pltpu.VMEM((C, D), jnp.float32),
        pltpu.VMEM((C, D), jnp.float32),
        pltpu.SemaphoreType.DMA,
    ],
)
def _batch_diff_sc(table_hbm, g0_hbm, g1_hbm,
                   out0, out1, out2, out3,
                   idx0_v, idx1_v, rows0, rows1, sem):
    wid = lax.axis_index("s") * NC + lax.axis_index("c")
    outs = (out0, out1, out2, out3)

    for l in range(LEVELS):
        out_l = outs[l]

        def round_body(t, _, out_l=out_l, l=l):
            chunk = t * NW + wid

            @pl.when(chunk < CHUNKS_PER_LEVEL)
            def _():
                gbase = l * NPAIR + chunk * C
                pltpu.sync_copy(g0_hbm.at[pl.ds(gbase, C)], idx0_v)
                pltpu.sync_copy(g1_hbm.at[pl.ds(gbase, C)], idx1_v)
                cp0 = pltpu.async_copy(table_hbm.at[idx0_v], rows0, sem)
                cp1 = pltpu.async_copy(table_hbm.at[idx1_v], rows1, sem)
                cp0.wait()
                cp1.wait()

                def row_body(r, carry):
                    for cc in range(D // LANES):
                        a = rows0[r, pl.ds(cc * LANES, LANES)]
                        b = rows1[r, pl.ds(cc * LANES, LANES)]
                        d = a - b
                        rows0[r, pl.ds(cc * LANES, LANES)] = d * d
                    return carry

                lax.fori_loop(0, C, row_body, 0)
                pltpu.sync_copy(rows0, out_l.at[pl.ds(chunk * C, C)])

            return 0

        lax.fori_loop(0, ROUNDS, round_body, 0)


def kernel(pyramid):
    table = pyramid.reshape(LEVELS * BATCH, D)
    return tuple(_batch_diff_sc(table, G0, G1))


# speedup vs baseline: 1.2217x; 1.2217x over previous
import functools

import jax
import jax.numpy as jnp
import numpy as np
from jax import lax
from jax.experimental import pallas as pl
from jax.experimental.pallas import tpu as pltpu
from jax.experimental.pallas import tpu_sc as plsc

LEVELS = 4
BATCH = 128
D = 1024
NPAIR = 8128
P_EXP = 2

NC = 2
NS = 16
NW = NC * NS
LANES = 16

C = 16
CPL = NPAIR // C
NOUTER = (CPL + 2 * NW - 1) // (2 * NW)

_i0, _i1 = np.triu_indices(n=BATCH, k=1)
G0 = np.concatenate([l * BATCH + _i0 for l in range(LEVELS)]).astype(np.int32)
G1 = np.concatenate([l * BATCH + _i1 for l in range(LEVELS)]).astype(np.int32)

_mesh = plsc.VectorSubcoreMesh(core_axis_name="c", subcore_axis_name="s")


@functools.partial(
    pl.kernel,
    mesh=_mesh,
    out_type=[jax.ShapeDtypeStruct((NPAIR, D), jnp.float32)
              for _ in range(LEVELS)],
    scratch_types=[
        pltpu.VMEM((NPAIR,), jnp.int32),
        pltpu.VMEM((NPAIR,), jnp.int32),
        pltpu.VMEM((C, D), jnp.float32),
        pltpu.VMEM((C, D), jnp.float32),
        pltpu.VMEM((C, D), jnp.float32),
        pltpu.VMEM((C, D), jnp.float32),
        pltpu.VMEM((C, D), jnp.float32),
        pltpu.VMEM((C, D), jnp.float32),
        pltpu.SemaphoreType.DMA,
        pltpu.SemaphoreType.DMA,
        pltpu.SemaphoreType.DMA,
        pltpu.SemaphoreType.DMA,
    ],
)
def _batch_diff_sc(table_hbm, g0_hbm, g1_hbm,
                   out0, out1, out2, out3,
                   idxa0, idxa1,
                   r0a, r0b, r1a, r1b, oba, obb,
                   sga, sgb, soa, sob):
    wid = lax.axis_index("s") * NC + lax.axis_index("c")
    outs = (out0, out1, out2, out3)
    rows0 = (r0a, r0b)
    rows1 = (r1a, r1b)
    ob = (oba, obb)
    sg = (sga, sgb)
    so = (soa, sob)

    nr = (CPL - 1 - wid) // NW + 1

    def issue_gather(chunk, s):
        base = chunk * C
        pltpu.async_copy(table_hbm.at[idxa0.at[pl.ds(base, C)]], rows0[s],
                         sg[s])
        pltpu.async_copy(table_hbm.at[idxa1.at[pl.ds(base, C)]], rows1[s],
                         sg[s])

    def wait_gather(s):
        pltpu.make_async_copy(table_hbm.at[pl.ds(0, C)], rows0[s],
                              sg[s]).wait()
        pltpu.make_async_copy(table_hbm.at[pl.ds(0, C)], rows1[s],
                              sg[s]).wait()

    for l in range(LEVELS):
        out_l = outs[l]
        pltpu.sync_copy(g0_hbm.at[pl.ds(l * NPAIR, NPAIR)], idxa0)
        pltpu.sync_copy(g1_hbm.at[pl.ds(l * NPAIR, NPAIR)], idxa1)

        issue_gather(wid, 0)

        def outer_body(g, _, out_l=out_l):
            for b in range(2):
                t = 2 * g + b

                @pl.when(t < nr)
                def _(t=t, b=b):
                    chunk = t * NW + wid
                    wait_gather(b)

                    @pl.when(t + 1 < nr)
                    def _():
                        issue_gather((t + 1) * NW + wid, 1 - b)

                    @pl.when(t >= 2)
                    def _():
                        pltpu.make_async_copy(
                            ob[b], out_l.at[pl.ds(0, C)], so[b]).wait()

                    def row_body(r, carry):
                        for cc in range(D // LANES):
                            sl = pl.ds(cc * LANES, LANES)
                            d = rows0[b][r, sl] - rows1[b][r, sl]
                            ob[b][r, sl] = d * d
                        return carry

                    lax.fori_loop(0, C, row_body, 0)
                    pltpu.async_copy(ob[b], out_l.at[pl.ds(chunk * C, C)],
                                     so[b])

            return 0

        lax.fori_loop(0, NOUTER, outer_body, 0)

        for b in range(2):
            pltpu.make_async_copy(ob[b], out_l.at[pl.ds(0, C)], so[b]).wait()


def kernel(pyramid):
    table = pyramid.reshape(LEVELS * BATCH, D)
    return tuple(_batch_diff_sc(table, G0, G1))
